# baseline (speedup 1.0000x reference)
import functools

import jax
import jax.numpy as jnp
from jax import lax
from jax.experimental import pallas as pl
from jax.experimental.pallas import tpu as pltpu
from jax.experimental.pallas import tpu_sc as plsc

CHUNK = 128
USED_ROWS = 500000


@functools.cache
def _build(total_nodes: int, embed_dim: int, batch_n: int):
  info = plsc.get_sparse_core_info()
  nw = info.num_cores * info.num_subcores
  b_per_w = batch_n // nw
  n_chunks = b_per_w // CHUNK
  mesh = plsc.VectorSubcoreMesh(core_axis_name="c", subcore_axis_name="s")

  @functools.partial(
      pl.kernel,
      mesh=mesh,
      out_type=jax.ShapeDtypeStruct((batch_n, embed_dim), jnp.float32),
      scratch_types=[
          pltpu.VMEM((n_chunks, CHUNK), jnp.int32),
          pltpu.VMEM((b_per_w, embed_dim), jnp.float32),
          pltpu.SemaphoreType.DMA,
      ],
      compiler_params=pltpu.CompilerParams(use_tc_tiling_on_sc=False),
  )
  def gather_kernel(table_hbm, idx_hbm, out_hbm, idx_v, rows_v, sem):
    wid = lax.axis_index("s") * info.num_cores + lax.axis_index("c")
    base = wid * b_per_w
    pltpu.sync_copy(idx_hbm.at[wid], idx_v)
    copies = [
        pltpu.async_copy(
            table_hbm.at[idx_v.at[j]],
            rows_v.at[pl.ds(j * CHUNK, CHUNK)],
            sem,
        )
        for j in range(n_chunks)
    ]
    for c in copies:
      c.wait()
    pltpu.sync_copy(rows_v, out_hbm.at[pl.ds(base, b_per_w)])

  return gather_kernel, nw, n_chunks


def kernel(W, batch):
  total_nodes, embed_dim = W.shape
  (batch_n,) = batch.shape
  used = lax.dynamic_slice_in_dim(W, 0, USED_ROWS, axis=0)
  gather_kernel, nw, n_chunks = _build(total_nodes, embed_dim, batch_n)
  idx = batch.astype(jnp.int32).reshape(nw, n_chunks, CHUNK)
  return gather_kernel(used, idx)


# speedup vs baseline: 1.1284x; 1.1284x over previous
import functools

import jax
import jax.numpy as jnp
from jax import lax
from jax.experimental import pallas as pl
from jax.experimental.pallas import tpu as pltpu
from jax.experimental.pallas import tpu_sc as plsc

NSEM = 4


@functools.cache
def _build(total_nodes: int, embed_dim: int, batch_n: int):
  info = plsc.get_sparse_core_info()
  nw = info.num_cores * info.num_subcores
  b_per_w = batch_n // nw
  mesh = plsc.VectorSubcoreMesh(core_axis_name="c", subcore_axis_name="s")

  @functools.partial(
      pl.kernel,
      mesh=mesh,
      out_type=jax.ShapeDtypeStruct((batch_n, embed_dim), jnp.float32),
      scratch_types=[
          pltpu.VMEM((b_per_w,), jnp.int32),
          pltpu.VMEM((b_per_w, embed_dim), jnp.float32),
          [pltpu.SemaphoreType.DMA] * NSEM,
      ],
  )
  def gather_kernel(table_hbm, idx_hbm, out_hbm, idx_v, rows_v, sems):
    wid = lax.axis_index("s") * info.num_cores + lax.axis_index("c")
    base = wid * b_per_w
    pltpu.sync_copy(idx_hbm.at[pl.ds(base, b_per_w)], idx_v)

    def fire(g, carry):
      vec = idx_v[pl.ds(g * 16, 16)]
      for lane in range(16):
        pltpu.async_copy(
            table_hbm.at[pl.ds(vec[lane], 1)],
            rows_v.at[pl.ds(g * 16 + lane, 1)],
            sems[lane % NSEM],
        )
      return carry

    lax.fori_loop(0, b_per_w // 16, fire, 0)

    def drain(k, carry):
      for s in range(NSEM):
        pltpu.make_async_copy(
            table_hbm.at[pl.ds(0, 1)], rows_v.at[pl.ds(0, 1)], sems[s]
        ).wait()
      return carry

    lax.fori_loop(0, b_per_w // NSEM, drain, 0)
    pltpu.sync_copy(rows_v, out_hbm.at[pl.ds(base, b_per_w)])

  return gather_kernel, nw


def kernel(W, batch):
  total_nodes, embed_dim = W.shape
  (batch_n,) = batch.shape
  gather_kernel, _ = _build(total_nodes, embed_dim, batch_n)
  return gather_kernel(W, batch.astype(jnp.int32))
